# baseline (speedup 1.0000x reference)
import functools

import jax
import jax.numpy as jnp
from jax import lax
from jax.experimental import pallas as pl
from jax.experimental.pallas import tpu as pltpu
from jax.experimental.pallas import tpu_sc as plsc

N = 512
E = 16384
F = 128
EA = 16
H = 64
CB = 5
W = 128

NTILE = 32
EPT = E // NTILE
NHALF = 2
HEPT = EPT // NHALF
NCHUNK = 2
ECH = HEPT // NCHUNK


def _tc_a_body(x_ref, ea_ref, wl_ref, w1a_ref, w1b_ref, b11_ref, w22_ref,
               b22_ref, h_ref, ga_ref, eb_ref, nb_ref):
    i = pl.program_id(0)

    @pl.when(i == 0)
    def _():
        h = jnp.maximum(jnp.dot(x_ref[...], wl_ref[...],
                                preferred_element_type=jnp.float32), 0.0)
        h_ref[...] = h
        ga_ref[:, :H] = jnp.dot(h, w1a_ref[...],
                                preferred_element_type=jnp.float32)
        ga_ref[:, H:] = jnp.zeros((N, W - H), jnp.float32)

    eat = ea_ref[...]
    eb_ref[...] = jnp.dot(eat, w1b_ref[...],
                          preferred_element_type=jnp.float32) + b11_ref[...]
    nb_ref[...] = jnp.dot(eat, w22_ref[...],
                          preferred_element_type=jnp.float32) + b22_ref[...]


def _tc_a(x, edge_attr, wl_t, w1a_t, w1b_t, b11, w22_t, b22):
    echunk = 2048
    grid = E // echunk
    return pl.pallas_call(
        _tc_a_body,
        grid=(grid,),
        in_specs=[
            pl.BlockSpec((N, F), lambda i: (0, 0)),
            pl.BlockSpec((echunk, EA), lambda i: (i, 0)),
            pl.BlockSpec((F, H), lambda i: (0, 0)),
            pl.BlockSpec((H, H), lambda i: (0, 0)),
            pl.BlockSpec((EA, H), lambda i: (0, 0)),
            pl.BlockSpec((1, H), lambda i: (0, 0)),
            pl.BlockSpec((EA, H), lambda i: (0, 0)),
            pl.BlockSpec((1, H), lambda i: (0, 0)),
        ],
        out_specs=[
            pl.BlockSpec((N, H), lambda i: (0, 0)),
            pl.BlockSpec((N, W), lambda i: (0, 0)),
            pl.BlockSpec((echunk, H), lambda i: (i, 0)),
            pl.BlockSpec((echunk, H), lambda i: (i, 0)),
        ],
        out_shape=[
            jax.ShapeDtypeStruct((N, H), jnp.float32),
            jax.ShapeDtypeStruct((N, W), jnp.float32),
            jax.ShapeDtypeStruct((E, H), jnp.float32),
            jax.ShapeDtypeStruct((E, H), jnp.float32),
        ],
    )(x, edge_attr, wl_t, w1a_t, w1b_t, b11, w22_t, b22)


def _sc_b_body(ga_hbm, eb_hbm, src_hbm, dst_hbm, out_hbm,
               idxs2, idxd2, rows, ebv, shared, sem):
    c = lax.axis_index("c")
    s = lax.axis_index("s")
    base = (c * 16 + s) * EPT

    def zb(i, _):
        for j in range(W // 16):
            rows[i, pl.ds(j * 16, 16)] = jnp.zeros((16,), jnp.float32)
        return 0
    lax.fori_loop(0, 32, zb, 0)
    pltpu.sync_copy(rows.at[pl.ds(0, 32)], shared.at[pl.ds(s * 32, 32)])
    plsc.subcore_barrier()

    for hh in range(NHALF):
        hbase = base + hh * HEPT
        for k in range(NCHUNK):
            pltpu.sync_copy(src_hbm.at[pl.ds(hbase + k * ECH, ECH)],
                            idxs2.at[k])
            pltpu.sync_copy(dst_hbm.at[pl.ds(hbase + k * ECH, ECH)],
                            idxd2.at[k])

        for k in range(NCHUNK):
            pltpu.async_copy(ga_hbm.at[idxs2.at[k]],
                             rows.at[pl.ds(k * ECH, ECH)], sem).wait()
        pltpu.sync_copy(eb_hbm.at[pl.ds(hbase, HEPT)], ebv)

        def body(e, _):
            for j in range(H // 16):
                sl = pl.ds(j * 16, 16)
                rows[e, sl] = jnp.maximum(rows[e, sl] + ebv[e, sl], 0.0)
            return 0
        lax.fori_loop(0, HEPT, body, 0)

        for k in range(NCHUNK):
            pltpu.sync_copy(rows.at[pl.ds(k * ECH, ECH)],
                            shared.at[idxd2.at[k]], add=True)
    plsc.subcore_barrier()
    pltpu.sync_copy(shared.at[pl.ds(s * 32, 32)],
                    out_hbm.at[c, pl.ds(s * 32, 32)])


def _sc_b(ga, eb, src, dst):
    mesh = plsc.VectorSubcoreMesh(core_axis_name="c", subcore_axis_name="s")
    kern = functools.partial(
        pl.kernel, mesh=mesh,
        out_type=jax.ShapeDtypeStruct((2, N, W), jnp.float32),
        scratch_types=[
            pltpu.VMEM((NCHUNK, ECH), jnp.int32),
            pltpu.VMEM((NCHUNK, ECH), jnp.int32),
            pltpu.VMEM((HEPT, W), jnp.float32),
            pltpu.VMEM((HEPT, H), jnp.float32),
            pltpu.VMEM_SHARED((N, W), jnp.float32),
            pltpu.SemaphoreType.DMA,
        ],
    )(_sc_b_body)
    return kern(ga, eb, src, dst)


def _tc_c_body(aggp_ref, h_ref, w12a_ref, w12b_ref, b12_ref, w23_ref,
               b23_ref, wg_ref, asrc_ref, adst_ref,
               h2_ref, naxp_ref, asrc_o, adst_o, aself_o):
    agg = aggp_ref[0, :, :H] + aggp_ref[1, :, :H]
    h2 = jnp.maximum(
        jnp.dot(agg, w12a_ref[...], preferred_element_type=jnp.float32)
        + jnp.dot(h_ref[...], w12b_ref[...], preferred_element_type=jnp.float32)
        + b12_ref[...], 0.0)
    h2_ref[...] = h2
    naxp_ref[:, :H] = jnp.dot(h2, w23_ref[...],
                              preferred_element_type=jnp.float32) + b23_ref[...]
    xp = jnp.dot(h2, wg_ref[...], preferred_element_type=jnp.float32)
    naxp_ref[:, H:] = xp
    a_src = jnp.sum(xp * asrc_ref[...], axis=1)
    a_dst = jnp.sum(xp * adst_ref[...], axis=1)
    z = a_src + a_dst
    a_self = jnp.maximum(z, 0.2 * z)
    asrc_o[...] = a_src
    adst_o[...] = a_dst
    aself_o[...] = a_self


def _tc_c(aggp, h, w12a_t, w12b_t, b12, w23_t, b23, wg_t, att_src, att_dst):
    return pl.pallas_call(
        _tc_c_body,
        out_shape=[
            jax.ShapeDtypeStruct((N, H), jnp.float32),
            jax.ShapeDtypeStruct((N, W), jnp.float32),
            jax.ShapeDtypeStruct((N,), jnp.float32),
            jax.ShapeDtypeStruct((N,), jnp.float32),
            jax.ShapeDtypeStruct((N,), jnp.float32),
        ],
    )(aggp, h, w12a_t, w12b_t, b12, w23_t, b23, wg_t,
      att_src.reshape(1, H), att_dst.reshape(1, H))


def _sc_dm_body(naxp_hbm, nb_hbm, asrc_hbm, adst_hbm, aself_hbm,
                src_hbm, dst_hbm,
                ocomb, oden,
                idxs2, idxd2, idxsf, idxdf, trows, nbv,
                asrc_v, adst_v, aself_v, ea_v, den_loc, dstage_v, dout_v,
                sh_comb, sh_dstage, sem):
    c = lax.axis_index("c")
    s = lax.axis_index("s")
    base = (c * 16 + s) * EPT
    iota16 = lax.iota(jnp.int32, 16)
    zeros16i = jnp.zeros((16,), jnp.int32)
    zeros16f = jnp.zeros((16,), jnp.float32)
    lane0 = iota16 == 0

    def zb(i, _):
        row = zeros16i + i
        for j in range(W // 16):
            plsc.store_scatter(trows, [row, iota16 + j * 16], zeros16f)
        return 0
    lax.fori_loop(0, 32, zb, 0)
    pltpu.sync_copy(trows.at[pl.ds(0, 32)], sh_comb.at[pl.ds(s * 32, 32)])

    def zd(i, _):
        den_loc[pl.ds(i * 16, 16)] = zeros16f
        return 0
    lax.fori_loop(0, N // 16, zd, 0)

    pltpu.sync_copy(asrc_hbm, asrc_v)
    pltpu.sync_copy(adst_hbm, adst_v)
    pltpu.sync_copy(aself_hbm, aself_v)
    pltpu.sync_copy(src_hbm.at[pl.ds(base, EPT)], idxsf)
    pltpu.sync_copy(dst_hbm.at[pl.ds(base, EPT)], idxdf)
    plsc.subcore_barrier()

    def gbody(g, _):
        sl = pl.ds(g * 16, 16)
        iv_s = idxsf[sl]
        iv_d = idxdf[sl]
        av = plsc.load_gather(asrc_v, [iv_s])
        qv = plsc.load_gather(adst_v, [iv_d])
        cv = plsc.load_gather(aself_v, [iv_d])
        sm = av + qv
        ae = jnp.maximum(sm, 0.2 * sm)
        ea_v[sl] = jnp.exp(ae - cv)
        return 0
    lax.fori_loop(0, EPT // 16, gbody, 0)

    def dbody(e, _):
        eab = plsc.load_gather(ea_v, [zeros16i + e])
        dstb = plsc.load_gather(idxdf, [zeros16i + e])
        plsc.addupdate_scatter(den_loc, [dstb], eab, mask=lane0)
        return 0
    lax.fori_loop(0, EPT, dbody, 0)

    for hh in range(NHALF):
        hbase = base + hh * HEPT
        for k in range(NCHUNK):
            pltpu.sync_copy(src_hbm.at[pl.ds(hbase + k * ECH, ECH)],
                            idxs2.at[k])
            pltpu.sync_copy(dst_hbm.at[pl.ds(hbase + k * ECH, ECH)],
                            idxd2.at[k])
        pltpu.sync_copy(nb_hbm.at[pl.ds(hbase, HEPT)], nbv)

        for k in range(NCHUNK):
            pltpu.async_copy(naxp_hbm.at[idxs2.at[k]],
                             trows.at[pl.ds(k * ECH, ECH)], sem).wait()

        def ebody(e, _):
            row = zeros16i + e
            eab = plsc.load_gather(ea_v, [zeros16i + (hh * HEPT + e)])
            for j in range(H // 16):
                col = iota16 + j * 16
                v = plsc.load_gather(trows, [row, col])
                nbr = plsc.load_gather(nbv, [row, col])
                plsc.store_scatter(trows, [row, col], v * nbr)
            for j in range(H // 16, W // 16):
                col = iota16 + j * 16
                v = plsc.load_gather(trows, [row, col])
                plsc.store_scatter(trows, [row, col], v * eab)
            return 0
        lax.fori_loop(0, HEPT, ebody, 0)

        for k in range(NCHUNK):
            pltpu.sync_copy(trows.at[pl.ds(k * ECH, ECH)],
                            sh_comb.at[idxd2.at[k]], add=True)

    pltpu.sync_copy(den_loc, sh_dstage.at[s])
    plsc.subcore_barrier()

    osl = pl.ds(s * 32, 32)
    pltpu.sync_copy(sh_comb.at[osl], ocomb.at[c, osl])

    for t in range(16):
        pltpu.sync_copy(sh_dstage.at[t, pl.ds(s * 32, 32)],
                        dstage_v.at[pl.ds(t * 32, 32)])

    def dsum(j, _):
        acc = jnp.zeros((16,), jnp.float32)
        for t in range(16):
            acc = acc + dstage_v[pl.ds(t * 32 + j * 16, 16)]
        dout_v[pl.ds(j * 16, 16)] = acc
        return 0
    lax.fori_loop(0, 2, dsum, 0)
    pltpu.sync_copy(dout_v, oden.at[c, osl])


def _sc_dm(naxp, nb, asrc, adst, aself, src, dst):
    mesh = plsc.VectorSubcoreMesh(core_axis_name="c", subcore_axis_name="s")
    kern = functools.partial(
        pl.kernel, mesh=mesh,
        out_type=[
            jax.ShapeDtypeStruct((2, N, W), jnp.float32),
            jax.ShapeDtypeStruct((2, N), jnp.float32),
        ],
        scratch_types=[
            pltpu.VMEM((NCHUNK, ECH), jnp.int32),
            pltpu.VMEM((NCHUNK, ECH), jnp.int32),
            pltpu.VMEM((EPT,), jnp.int32),
            pltpu.VMEM((EPT,), jnp.int32),
            pltpu.VMEM((HEPT, W), jnp.float32),
            pltpu.VMEM((HEPT, H), jnp.float32),
            pltpu.VMEM((N,), jnp.float32),
            pltpu.VMEM((N,), jnp.float32),
            pltpu.VMEM((N,), jnp.float32),
            pltpu.VMEM((EPT,), jnp.float32),
            pltpu.VMEM((N,), jnp.float32),
            pltpu.VMEM((N,), jnp.float32),
            pltpu.VMEM((32,), jnp.float32),
            pltpu.VMEM_SHARED((N, W), jnp.float32),
            pltpu.VMEM_SHARED((16, N), jnp.float32),
            pltpu.SemaphoreType.DMA,
        ],
        compiler_params=pltpu.CompilerParams(needs_layout_passes=False),
    )(_sc_dm_body)
    return kern(naxp, nb, asrc, adst, aself, src, dst)


def _tc_e_body(combp_ref, denp_ref, h2_ref, naxp_ref, w23_ref,
               b23_ref, bg_ref, wp2_ref, wp3_ref, u_ref):
    agg2 = combp_ref[0, :, :H] + combp_ref[1, :, :H]
    xp = naxp_ref[:, H:]
    num = combp_ref[0, :, H:] + combp_ref[1, :, H:] + xp
    den = denp_ref[0, :] + denp_ref[1, :] + 1.0
    local = jnp.dot(agg2 * h2_ref[...], w23_ref[...],
                    preferred_element_type=jnp.float32) + b23_ref[...]
    glob = num / (den + 1e-16)[:, None] + bg_ref[...]
    sarr = local + glob
    wc = jnp.dot(wp2_ref[...], wp3_ref[...],
                 preferred_element_type=jnp.float32)
    u_ref[...] = jnp.dot(sarr, wc, preferred_element_type=jnp.float32)


def _tc_e(combp, denp, h2, naxp, w23_t, b23, bg, wp2_t, wp3_t):
    return pl.pallas_call(
        _tc_e_body,
        out_shape=jax.ShapeDtypeStruct((N, CB), jnp.float32),
    )(combp, denp, h2, naxp, w23_t, b23, bg, wp2_t, wp3_t)


_XB3 = 8


def _tc_f_body(u_ref, ua_ref, out_ref):
    i = pl.program_id(0)
    u8 = u_ref[...]
    ua = ua_ref[...]
    t = u8.reshape(_XB3, 1, CB) + ua.reshape(1, N, CB)
    xg = lax.broadcasted_iota(jnp.int32, (_XB3, N, CB), 0) + _XB3 * i
    yg = lax.broadcasted_iota(jnp.int32, (_XB3, N, CB), 1)
    out_ref[...] = jnp.where(xg == yg, -1.0, t).reshape(_XB3 * N, CB)


def _tc_f(u):
    return pl.pallas_call(
        _tc_f_body,
        grid=(N // _XB3,),
        in_specs=[
            pl.BlockSpec((_XB3, CB), lambda i: (i, 0)),
            pl.BlockSpec((N, CB), lambda i: (0, 0)),
        ],
        out_specs=pl.BlockSpec((_XB3 * N, CB), lambda i: (i, 0)),
        out_shape=jax.ShapeDtypeStruct((N * N, CB), jnp.float32),
    )(u, u)


def kernel(x, edge_index, edge_attr, W_lin, W1_1, b1_1, W1_2, b1_2,
           W2_2, b2_2, W2_3, b2_3, Wg, att_src, att_dst, bg, Wp2, Wp3):
    src = edge_index[0]
    dst = edge_index[1]

    wl_t = W_lin.T
    w1a_t = W1_1[:, :H].T
    w1b_t = W1_1[:, H:].T
    w12a_t = W1_2[:, :H].T
    w12b_t = W1_2[:, H:].T
    w22_t = W2_2.T
    w23_t = W2_3.T
    wg_t = Wg.T
    wp2_t = Wp2.T
    wp3_t = Wp3.T

    h, ga, eb, nb = _tc_a(x, edge_attr, wl_t, w1a_t, w1b_t,
                          b1_1.reshape(1, H), w22_t, b2_2.reshape(1, H))
    aggp = _sc_b(ga, eb, src, dst)
    h2, naxp, asrc, adst, aself = _tc_c(
        aggp, h, w12a_t, w12b_t, b1_2.reshape(1, H),
        w23_t, b2_3.reshape(1, H), wg_t, att_src, att_dst)
    combp, denp = _sc_dm(naxp, nb, asrc, adst, aself, src, dst)
    u = _tc_e(combp, denp, h2, naxp, w23_t, b2_3.reshape(1, H),
              bg.reshape(1, H), wp2_t, wp3_t)
    return _tc_f(u)


# speedup vs baseline: 1.0516x; 1.0516x over previous
import functools

import jax
import jax.numpy as jnp
from jax import lax
from jax.experimental import pallas as pl
from jax.experimental.pallas import tpu as pltpu
from jax.experimental.pallas import tpu_sc as plsc

N = 512
E = 16384
F = 128
EA = 16
H = 64
CB = 5
W = 128

NTILE = 32
EPT = E // NTILE
NHALF = 2
HEPT = EPT // NHALF
NCHUNK = 2
ECH = HEPT // NCHUNK


def _tc_a_body(x_ref, ea_ref, wl_ref, w1a_ref, w1b_ref, b11_ref, w22_ref,
               b22_ref, h_ref, ga_ref, eb_ref, nb_ref):
    i = pl.program_id(0)

    @pl.when(i == 0)
    def _():
        h = jnp.maximum(jnp.dot(x_ref[...], wl_ref[...],
                                preferred_element_type=jnp.float32), 0.0)
        h_ref[...] = h
        ga_ref[:, :H] = jnp.dot(h, w1a_ref[...],
                                preferred_element_type=jnp.float32)
        ga_ref[:, H:] = jnp.zeros((N, W - H), jnp.float32)

    eat = ea_ref[...]
    eb_ref[...] = jnp.dot(eat, w1b_ref[...],
                          preferred_element_type=jnp.float32) + b11_ref[...]
    nb_ref[...] = jnp.dot(eat, w22_ref[...],
                          preferred_element_type=jnp.float32) + b22_ref[...]


def _tc_a(x, edge_attr, wl_t, w1a_t, w1b_t, b11, w22_t, b22):
    echunk = 2048
    grid = E // echunk
    return pl.pallas_call(
        _tc_a_body,
        grid=(grid,),
        in_specs=[
            pl.BlockSpec((N, F), lambda i: (0, 0)),
            pl.BlockSpec((echunk, EA), lambda i: (i, 0)),
            pl.BlockSpec((F, H), lambda i: (0, 0)),
            pl.BlockSpec((H, H), lambda i: (0, 0)),
            pl.BlockSpec((EA, H), lambda i: (0, 0)),
            pl.BlockSpec((1, H), lambda i: (0, 0)),
            pl.BlockSpec((EA, H), lambda i: (0, 0)),
            pl.BlockSpec((1, H), lambda i: (0, 0)),
        ],
        out_specs=[
            pl.BlockSpec((N, H), lambda i: (0, 0)),
            pl.BlockSpec((N, W), lambda i: (0, 0)),
            pl.BlockSpec((echunk, H), lambda i: (i, 0)),
            pl.BlockSpec((echunk, H), lambda i: (i, 0)),
        ],
        out_shape=[
            jax.ShapeDtypeStruct((N, H), jnp.float32),
            jax.ShapeDtypeStruct((N, W), jnp.float32),
            jax.ShapeDtypeStruct((E, H), jnp.float32),
            jax.ShapeDtypeStruct((E, H), jnp.float32),
        ],
    )(x, edge_attr, wl_t, w1a_t, w1b_t, b11, w22_t, b22)


def _sc_b_body(ga_hbm, eb_hbm, src_hbm, dst_hbm, out_hbm,
               idxs2, idxd2, rows, ebv, shared, sem):
    c = lax.axis_index("c")
    s = lax.axis_index("s")
    base = (c * 16 + s) * EPT

    def zb(i, _):
        for j in range(W // 16):
            rows[i, pl.ds(j * 16, 16)] = jnp.zeros((16,), jnp.float32)
        return 0
    lax.fori_loop(0, 32, zb, 0)
    pltpu.sync_copy(rows.at[pl.ds(0, 32)], shared.at[pl.ds(s * 32, 32)])
    plsc.subcore_barrier()

    for hh in range(NHALF):
        hbase = base + hh * HEPT
        for k in range(NCHUNK):
            pltpu.sync_copy(src_hbm.at[pl.ds(hbase + k * ECH, ECH)],
                            idxs2.at[k])
            pltpu.sync_copy(dst_hbm.at[pl.ds(hbase + k * ECH, ECH)],
                            idxd2.at[k])

        for k in range(NCHUNK):
            pltpu.async_copy(ga_hbm.at[idxs2.at[k]],
                             rows.at[pl.ds(k * ECH, ECH)], sem).wait()
        pltpu.sync_copy(eb_hbm.at[pl.ds(hbase, HEPT)], ebv)

        def body(e, _):
            for j in range(H // 16):
                sl = pl.ds(j * 16, 16)
                rows[e, sl] = jnp.maximum(rows[e, sl] + ebv[e, sl], 0.0)
            return 0
        lax.fori_loop(0, HEPT, body, 0)

        for k in range(NCHUNK):
            pltpu.sync_copy(rows.at[pl.ds(k * ECH, ECH)],
                            shared.at[idxd2.at[k]], add=True)
    plsc.subcore_barrier()
    pltpu.sync_copy(shared.at[pl.ds(s * 32, 32)],
                    out_hbm.at[c, pl.ds(s * 32, 32)])


def _sc_b(ga, eb, src, dst):
    mesh = plsc.VectorSubcoreMesh(core_axis_name="c", subcore_axis_name="s")
    kern = functools.partial(
        pl.kernel, mesh=mesh,
        out_type=jax.ShapeDtypeStruct((2, N, W), jnp.float32),
        scratch_types=[
            pltpu.VMEM((NCHUNK, ECH), jnp.int32),
            pltpu.VMEM((NCHUNK, ECH), jnp.int32),
            pltpu.VMEM((HEPT, W), jnp.float32),
            pltpu.VMEM((HEPT, H), jnp.float32),
            pltpu.VMEM_SHARED((N, W), jnp.float32),
            pltpu.SemaphoreType.DMA,
        ],
    )(_sc_b_body)
    return kern(ga, eb, src, dst)


def _tc_c_body(aggp_ref, h_ref, w12a_ref, w12b_ref, b12_ref, w23_ref,
               b23_ref, wg_ref, asrc_ref, adst_ref,
               h2_ref, naxp_ref, asrc_o, adst_o, aself_o):
    agg = aggp_ref[0, :, :H] + aggp_ref[1, :, :H]
    h2 = jnp.maximum(
        jnp.dot(agg, w12a_ref[...], preferred_element_type=jnp.float32)
        + jnp.dot(h_ref[...], w12b_ref[...], preferred_element_type=jnp.float32)
        + b12_ref[...], 0.0)
    h2_ref[...] = h2
    naxp_ref[:, :H] = jnp.dot(h2, w23_ref[...],
                              preferred_element_type=jnp.float32) + b23_ref[...]
    xp = jnp.dot(h2, wg_ref[...], preferred_element_type=jnp.float32)
    naxp_ref[:, H:] = xp
    a_src = jnp.sum(xp * asrc_ref[...], axis=1)
    a_dst = jnp.sum(xp * adst_ref[...], axis=1)
    z = a_src + a_dst
    a_self = jnp.maximum(z, 0.2 * z)
    asrc_o[...] = a_src
    adst_o[...] = a_dst
    aself_o[...] = a_self


def _tc_c(aggp, h, w12a_t, w12b_t, b12, w23_t, b23, wg_t, att_src, att_dst):
    return pl.pallas_call(
        _tc_c_body,
        out_shape=[
            jax.ShapeDtypeStruct((N, H), jnp.float32),
            jax.ShapeDtypeStruct((N, W), jnp.float32),
            jax.ShapeDtypeStruct((N,), jnp.float32),
            jax.ShapeDtypeStruct((N,), jnp.float32),
            jax.ShapeDtypeStruct((N,), jnp.float32),
        ],
    )(aggp, h, w12a_t, w12b_t, b12, w23_t, b23, wg_t,
      att_src.reshape(1, H), att_dst.reshape(1, H))


def _sc_d1_body(asrc_hbm, adst_hbm, aself_hbm, src_hbm, dst_hbm,
                oea, oden,
                idxsf, idxdf, asrc_v, adst_v, aself_v, ea_v, den_loc,
                dstage_v, dout_v, sh_dstage):
    c = lax.axis_index("c")
    s = lax.axis_index("s")
    base = (c * 16 + s) * EPT

    def zd(i, _):
        den_loc[pl.ds(i * 16, 16)] = jnp.zeros((16,), jnp.float32)
        return 0
    lax.fori_loop(0, N // 16, zd, 0)

    pltpu.sync_copy(asrc_hbm, asrc_v)
    pltpu.sync_copy(adst_hbm, adst_v)
    pltpu.sync_copy(aself_hbm, aself_v)
    pltpu.sync_copy(src_hbm.at[pl.ds(base, EPT)], idxsf)
    pltpu.sync_copy(dst_hbm.at[pl.ds(base, EPT)], idxdf)

    def gbody(g, _):
        sl = pl.ds(g * 16, 16)
        iv_s = idxsf[sl]
        iv_d = idxdf[sl]
        av = plsc.load_gather(asrc_v, [iv_s])
        qv = plsc.load_gather(adst_v, [iv_d])
        cv = plsc.load_gather(aself_v, [iv_d])
        sm = av + qv
        ae = jnp.maximum(sm, 0.2 * sm)
        ea_v[sl] = jnp.exp(ae - cv)
        return 0
    lax.fori_loop(0, EPT // 16, gbody, 0)

    lane0 = lax.iota(jnp.int32, 16) == 0
    zeros16i = jnp.zeros((16,), jnp.int32)

    def ebody(e, _):
        eab = plsc.load_gather(ea_v, [zeros16i + e])
        dstb = plsc.load_gather(idxdf, [zeros16i + e])
        plsc.addupdate_scatter(den_loc, [dstb], eab, mask=lane0)
        return 0
    lax.fori_loop(0, EPT, ebody, 0)

    pltpu.sync_copy(ea_v, oea.at[pl.ds(base, EPT)])
    pltpu.sync_copy(den_loc, sh_dstage.at[s])
    plsc.subcore_barrier()

    for t in range(16):
        pltpu.sync_copy(sh_dstage.at[t, pl.ds(s * 32, 32)],
                        dstage_v.at[pl.ds(t * 32, 32)])

    def dsum(j, _):
        acc = jnp.zeros((16,), jnp.float32)
        for t in range(16):
            acc = acc + dstage_v[pl.ds(t * 32 + j * 16, 16)]
        dout_v[pl.ds(j * 16, 16)] = acc
        return 0
    lax.fori_loop(0, 2, dsum, 0)
    pltpu.sync_copy(dout_v, oden.at[c, pl.ds(s * 32, 32)])


def _sc_d1(asrc, adst, aself, src, dst):
    mesh = plsc.VectorSubcoreMesh(core_axis_name="c", subcore_axis_name="s")
    kern = functools.partial(
        pl.kernel, mesh=mesh,
        out_type=[
            jax.ShapeDtypeStruct((E,), jnp.float32),
            jax.ShapeDtypeStruct((2, N), jnp.float32),
        ],
        scratch_types=[
            pltpu.VMEM((EPT,), jnp.int32),
            pltpu.VMEM((EPT,), jnp.int32),
            pltpu.VMEM((N,), jnp.float32),
            pltpu.VMEM((N,), jnp.float32),
            pltpu.VMEM((N,), jnp.float32),
            pltpu.VMEM((EPT,), jnp.float32),
            pltpu.VMEM((N,), jnp.float32),
            pltpu.VMEM((N,), jnp.float32),
            pltpu.VMEM((32,), jnp.float32),
            pltpu.VMEM_SHARED((16, N), jnp.float32),
        ],
        compiler_params=pltpu.CompilerParams(needs_layout_passes=False),
    )(_sc_d1_body)
    return kern(asrc, adst, aself, src, dst)


def _sc_d2_body(naxp_hbm, nb_hbm, ea_hbm, src_hbm, dst_hbm,
                ocomb,
                idxs2, idxd2, trows, nbv, ea_v, sh_comb, sem):
    c = lax.axis_index("c")
    s = lax.axis_index("s")
    base = (c * 16 + s) * EPT

    def zb(i, _):
        for j in range(W // 16):
            trows[i, pl.ds(j * 16, 16)] = jnp.zeros((16,), jnp.float32)
        return 0
    lax.fori_loop(0, 32, zb, 0)
    pltpu.sync_copy(trows.at[pl.ds(0, 32)], sh_comb.at[pl.ds(s * 32, 32)])
    plsc.subcore_barrier()

    for hh in range(NHALF):
        hbase = base + hh * HEPT
        for k in range(NCHUNK):
            pltpu.sync_copy(src_hbm.at[pl.ds(hbase + k * ECH, ECH)],
                            idxs2.at[k])
            pltpu.sync_copy(dst_hbm.at[pl.ds(hbase + k * ECH, ECH)],
                            idxd2.at[k])

        pltpu.sync_copy(nb_hbm.at[pl.ds(hbase, HEPT)], nbv)
        pltpu.sync_copy(ea_hbm.at[pl.ds(hbase, HEPT)], ea_v)

        for k in range(NCHUNK):
            pltpu.async_copy(naxp_hbm.at[idxs2.at[k]],
                             trows.at[pl.ds(k * ECH, ECH)], sem).wait()

        def ebody(g, _):
            grp = ea_v[pl.ds(g * 16, 16)]
            for ll in range(16):
                e = g * 16 + ll
                eab = jnp.broadcast_to(grp[ll], (16,))
                for j in range(H // 16):
                    sl = pl.ds(j * 16, 16)
                    trows[e, sl] = trows[e, sl] * nbv[e, sl]
                for j in range(H // 16, W // 16):
                    sl = pl.ds(j * 16, 16)
                    trows[e, sl] = trows[e, sl] * eab
            return 0
        lax.fori_loop(0, HEPT // 16, ebody, 0)

        for k in range(NCHUNK):
            pltpu.sync_copy(trows.at[pl.ds(k * ECH, ECH)],
                            sh_comb.at[idxd2.at[k]], add=True)

    plsc.subcore_barrier()
    osl = pl.ds(s * 32, 32)
    pltpu.sync_copy(sh_comb.at[osl], ocomb.at[c, osl])


def _sc_d2(naxp, nb, ea, src, dst):
    mesh = plsc.VectorSubcoreMesh(core_axis_name="c", subcore_axis_name="s")
    kern = functools.partial(
        pl.kernel, mesh=mesh,
        out_type=jax.ShapeDtypeStruct((2, N, W), jnp.float32),
        scratch_types=[
            pltpu.VMEM((NCHUNK, ECH), jnp.int32),
            pltpu.VMEM((NCHUNK, ECH), jnp.int32),
            pltpu.VMEM((HEPT, W), jnp.float32),
            pltpu.VMEM((HEPT, H), jnp.float32),
            pltpu.VMEM((HEPT,), jnp.float32),
            pltpu.VMEM_SHARED((N, W), jnp.float32),
            pltpu.SemaphoreType.DMA,
        ],
    )(_sc_d2_body)
    return kern(naxp, nb, ea, src, dst)


def _tc_e_body(combp_ref, denp_ref, h2_ref, naxp_ref, w23_ref,
               b23_ref, bg_ref, wp2_ref, wp3_ref, u_ref):
    agg2 = combp_ref[0, :, :H] + combp_ref[1, :, :H]
    xp = naxp_ref[:, H:]
    num = combp_ref[0, :, H:] + combp_ref[1, :, H:] + xp
    den = denp_ref[0, :] + denp_ref[1, :] + 1.0
    local = jnp.dot(agg2 * h2_ref[...], w23_ref[...],
                    preferred_element_type=jnp.float32) + b23_ref[...]
    glob = num / (den + 1e-16)[:, None] + bg_ref[...]
    sarr = local + glob
    wc = jnp.dot(wp2_ref[...], wp3_ref[...],
                 preferred_element_type=jnp.float32)
    u_ref[...] = jnp.dot(sarr, wc, preferred_element_type=jnp.float32)


def _tc_e(combp, denp, h2, naxp, w23_t, b23, bg, wp2_t, wp3_t):
    return pl.pallas_call(
        _tc_e_body,
        out_shape=jax.ShapeDtypeStruct((N, CB), jnp.float32),
    )(combp, denp, h2, naxp, w23_t, b23, bg, wp2_t, wp3_t)


_XB3 = 8


def _tc_f_body(u_ref, ua_ref, out_ref):
    i = pl.program_id(0)
    u8 = u_ref[...]
    ua = ua_ref[...]
    t = u8.reshape(_XB3, 1, CB) + ua.reshape(1, N, CB)
    xg = lax.broadcasted_iota(jnp.int32, (_XB3, N, CB), 0) + _XB3 * i
    yg = lax.broadcasted_iota(jnp.int32, (_XB3, N, CB), 1)
    out_ref[...] = jnp.where(xg == yg, -1.0, t).reshape(_XB3 * N, CB)


def _tc_f(u):
    return pl.pallas_call(
        _tc_f_body,
        grid=(N // _XB3,),
        in_specs=[
            pl.BlockSpec((_XB3, CB), lambda i: (i, 0)),
            pl.BlockSpec((N, CB), lambda i: (0, 0)),
        ],
        out_specs=pl.BlockSpec((_XB3 * N, CB), lambda i: (i, 0)),
        out_shape=jax.ShapeDtypeStruct((N * N, CB), jnp.float32),
    )(u, u)


def kernel(x, edge_index, edge_attr, W_lin, W1_1, b1_1, W1_2, b1_2,
           W2_2, b2_2, W2_3, b2_3, Wg, att_src, att_dst, bg, Wp2, Wp3):
    src = edge_index[0]
    dst = edge_index[1]

    wl_t = W_lin.T
    w1a_t = W1_1[:, :H].T
    w1b_t = W1_1[:, H:].T
    w12a_t = W1_2[:, :H].T
    w12b_t = W1_2[:, H:].T
    w22_t = W2_2.T
    w23_t = W2_3.T
    wg_t = Wg.T
    wp2_t = Wp2.T
    wp3_t = Wp3.T

    h, ga, eb, nb = _tc_a(x, edge_attr, wl_t, w1a_t, w1b_t,
                          b1_1.reshape(1, H), w22_t, b2_2.reshape(1, H))
    aggp = _sc_b(ga, eb, src, dst)
    h2, naxp, asrc, adst, aself = _tc_c(
        aggp, h, w12a_t, w12b_t, b1_2.reshape(1, H),
        w23_t, b2_3.reshape(1, H), wg_t, att_src, att_dst)
    ea_w, denp = _sc_d1(asrc, adst, aself, src, dst)
    combp = _sc_d2(naxp, nb, ea_w, src, dst)
    u = _tc_e(combp, denp, h2, naxp, w23_t, b2_3.reshape(1, H),
              bg.reshape(1, H), wp2_t, wp3_t)
    return _tc_f(u)
